# baseline (speedup 1.0000x reference)
import functools

import jax
import jax.numpy as jnp
from jax import lax
from jax.experimental import pallas as pl
from jax.experimental.pallas import tpu as pltpu
from jax.experimental.pallas import tpu_sc as plsc

B = 16384
EPS = 1e-5

NC, NS, L = 2, 16, 16
NW = NC * NS
BPW = B // NW
NCHUNK = BPW // 128

_sc_mesh = plsc.VectorSubcoreMesh(core_axis_name="c", subcore_axis_name="s")


@functools.partial(
    pl.kernel,
    mesh=_sc_mesh,
    out_type=jax.ShapeDtypeStruct((B, 128), jnp.float32),
    scratch_types=[
        pltpu.VMEM((BPW,), jnp.float32),
        pltpu.VMEM((BPW,), jnp.float32),
        pltpu.VMEM((NCHUNK, 128), jnp.int32),
        pltpu.VMEM((BPW, 128), jnp.float32),
        pltpu.SemaphoreType.DMA,
    ],
)
def _sc_gather(sidx_hbm, fidx_hbm, combo_hbm, out_hbm,
               sidx_v, fidx_v, idx_v, rows_v, sem):
    wid = lax.axis_index("s") * NC + lax.axis_index("c")
    base = wid * BPW
    pltpu.sync_copy(sidx_hbm.at[pl.ds(base, BPW)], sidx_v)
    pltpu.sync_copy(fidx_hbm.at[pl.ds(base, BPW)], fidx_v)
    for i in range(BPW // L):
        s = sidx_v[pl.ds(i * L, L)].astype(jnp.int32)
        f = fidx_v[pl.ds(i * L, L)].astype(jnp.int32)
        idx_v[i // 8, pl.ds((i % 8) * L, L)] = s * 33 + f
    for j in range(NCHUNK):
        pltpu.async_copy(combo_hbm.at[idx_v.at[j]],
                         rows_v.at[pl.ds(j * 128, 128)], sem).wait()
    pltpu.sync_copy(rows_v, out_hbm.at[pl.ds(base, BPW)])


def _mlp_body(X_ref, pairs_ref, W1a_ref, W1bc_ref, b1_ref,
              g1_ref, be1_ref, W2_ref, b2_ref, g2_ref, be2_ref,
              W3_ref, b3_ref, out_ref):
    X = X_ref[...]
    h = (jnp.dot(X[:, 0:8], W1a_ref[...], preferred_element_type=jnp.float32)
         + jnp.dot(pairs_ref[...], W1bc_ref[...],
                   preferred_element_type=jnp.float32)
         + b1_ref[...])
    h = jnp.maximum(h, 0.0)
    mu = jnp.mean(h, axis=0, keepdims=True)
    var = jnp.mean((h - mu) * (h - mu), axis=0, keepdims=True)
    h = g1_ref[...] * (h - mu) * lax.rsqrt(var + EPS) + be1_ref[...]

    h = jnp.dot(h, W2_ref[...], preferred_element_type=jnp.float32) + b2_ref[...]
    h = jnp.maximum(h, 0.0)
    mu2 = jnp.mean(h, axis=0, keepdims=True)
    var2 = jnp.mean((h - mu2) * (h - mu2), axis=0, keepdims=True)
    h = g2_ref[...] * (h - mu2) * lax.rsqrt(var2 + EPS) + be2_ref[...]

    out_ref[...] = (jnp.dot(h, W3_ref[...], preferred_element_type=jnp.float32)
                    + b3_ref[...])


def kernel(X, family_table, store_table, W1, b1, g1, be1, W2, b2, g2, be2, W3, b3):
    sidx_f = X[:, 8]
    fidx_f = X[:, 9]
    fam_part = jnp.tile(family_table, (54, 1))
    sto_part = jnp.repeat(store_table, 33, axis=0)
    combo = jnp.concatenate(
        [fam_part, sto_part, jnp.zeros((1782, 98), jnp.float32)], axis=1)
    combo = jnp.concatenate([combo, jnp.zeros((2, 128), jnp.float32)], axis=0)

    pairs = _sc_gather(sidx_f, fidx_f, combo)

    W1a = W1[0:8]
    W1bc = jnp.concatenate(
        [W1[8:23], W1[23:38], jnp.zeros((98, 20), jnp.float32)], axis=0)
    args = (X, pairs, W1a, W1bc,
            b1.reshape(1, -1), g1.reshape(1, -1), be1.reshape(1, -1),
            W2, b2.reshape(1, -1), g2.reshape(1, -1), be2.reshape(1, -1),
            W3, b3.reshape(1, -1))
    return pl.pallas_call(
        _mlp_body,
        out_shape=jax.ShapeDtypeStruct((B, 1), jnp.float32),
    )(*args)


# speedup vs baseline: 1.9939x; 1.9939x over previous
import numpy as np

import jax
import jax.numpy as jnp
from jax import lax
from jax.experimental import pallas as pl

B = 16384
P = 4
RP = B // P
EPS = 1e-5

_SEL54 = np.zeros((10 * P, 54 * P), np.float32)
_SEL33 = np.zeros((10 * P, 33 * P), np.float32)
for _c in range(P):
    _SEL54[10 * _c + 8, 54 * _c:54 * _c + 54] = 1.0
    _SEL33[10 * _c + 9, 33 * _c:33 * _c + 33] = 1.0
_V54 = np.tile(np.arange(54, dtype=np.float32), P)[None, :]
_V33 = np.tile(np.arange(33, dtype=np.float32), P)[None, :]


def _blockdiag(w, n):
    cols = w.shape[1]
    return jnp.concatenate(
        [jnp.pad(w, ((0, 0), (cols * c, cols * (n - 1 - c)))) for c in range(n)],
        axis=0)


def _bn_scale_shift(h, g, be, width):
    m = jnp.mean(h, axis=0, keepdims=True)
    q = jnp.mean(h * h, axis=0, keepdims=True)
    mc = sum(m[:, width * c:width * (c + 1)] for c in range(P)) * (1.0 / P)
    qc = sum(q[:, width * c:width * (c + 1)] for c in range(P)) * (1.0 / P)
    var = qc - mc * mc
    scale = g * lax.rsqrt(var + EPS)
    shift = be - mc * scale
    return (jnp.concatenate([scale] * P, axis=1),
            jnp.concatenate([shift] * P, axis=1))


def _packed_body(Xp_ref, ft_ref, st_ref, W1b_ref, W1c_ref, sel54_ref, v54_ref,
                 sel33_ref, v33_ref, W1a_ref, b1_ref, g1_ref, be1_ref,
                 W2_ref, b2_ref, g2_ref, be2_ref, W3_ref, b3_ref, out_ref):
    Xp = Xp_ref[...]

    s_val = jnp.dot(Xp, sel54_ref[...], preferred_element_type=jnp.float32)
    f_val = jnp.dot(Xp, sel33_ref[...], preferred_element_type=jnp.float32)
    oh_s = (s_val == v54_ref[...]).astype(jnp.float32)
    oh_f = (f_val == v33_ref[...]).astype(jnp.float32)

    stW = jnp.dot(st_ref[...], W1c_ref[...], preferred_element_type=jnp.float32)
    ftW = jnp.dot(ft_ref[...], W1b_ref[...], preferred_element_type=jnp.float32)
    h = (jnp.dot(Xp, W1a_ref[...], preferred_element_type=jnp.float32)
         + jnp.dot(oh_s, _blockdiag(stW, P), preferred_element_type=jnp.float32)
         + jnp.dot(oh_f, _blockdiag(ftW, P), preferred_element_type=jnp.float32)
         + b1_ref[...])
    h = jnp.maximum(h, 0.0)
    scale, shift = _bn_scale_shift(h, g1_ref[...], be1_ref[...], 20)
    h = h * scale + shift

    h = jnp.dot(h, W2_ref[...], preferred_element_type=jnp.float32) + b2_ref[...]
    h = jnp.maximum(h, 0.0)
    scale2, shift2 = _bn_scale_shift(h, g2_ref[...], be2_ref[...], 10)
    h = h * scale2 + shift2

    out_ref[...] = (jnp.dot(h, W3_ref[...], preferred_element_type=jnp.float32)
                    + b3_ref[...])


def kernel(X, family_table, store_table, W1, b1, g1, be1, W2, b2, g2, be2, W3, b3):
    Xp = X.reshape(RP, 10 * P)
    W1a10 = jnp.concatenate([W1[0:8], jnp.zeros((2, 20), jnp.float32)], axis=0)
    W1a_exp = _blockdiag(W1a10, P)
    W2_exp = _blockdiag(W2, P)
    W3_exp = _blockdiag(W3, P)
    args = (Xp, family_table, store_table, W1[8:23], W1[23:38],
            jnp.asarray(_SEL54), jnp.asarray(_V54),
            jnp.asarray(_SEL33), jnp.asarray(_V33),
            W1a_exp,
            jnp.tile(b1.reshape(1, -1), (1, P)),
            g1.reshape(1, -1), be1.reshape(1, -1),
            W2_exp,
            jnp.tile(b2.reshape(1, -1), (1, P)),
            g2.reshape(1, -1), be2.reshape(1, -1),
            W3_exp,
            jnp.tile(b3.reshape(1, -1), (1, P)))
    out_p = pl.pallas_call(
        _packed_body,
        out_shape=jax.ShapeDtypeStruct((RP, P), jnp.float32),
    )(*args)
    return out_p.reshape(B, 1)
